# baseline (speedup 1.0000x reference)
import functools

import jax
import jax.numpy as jnp
from jax import lax
from jax.experimental import pallas as pl
from jax.experimental.pallas import tpu as pltpu
from jax.experimental.pallas import tpu_sc as plsc

_B = 16384
_EMB = 16
_HID = 32
_NCTX = 8
_NC = 2
_NS = 16
_NW = _NC * _NS
_BPW = _B // _NW
_CH = 256
_NCHUNK = _BPW // _CH
_ROWW = _EMB + _EMB + _NCTX
_L = 16
_NWTS = _ROWW * _HID + _HID + _HID + 16


def _splat(v, k):
    return lax.gather(
        v, jnp.full((_L, 1), k, jnp.int32),
        dimension_numbers=lax.GatherDimensionNumbers(
            offset_dims=(), collapsed_slice_dims=(0,), start_index_map=(0,)),
        slice_sizes=(1,),
        mode=lax.GatherScatterMode.PROMISE_IN_BOUNDS)


def _fused_sc(tu, ti, users, items, ctx_flat, wts):
    mesh = plsc.VectorSubcoreMesh(core_axis_name="c", subcore_axis_name="s")

    @functools.partial(
        pl.kernel,
        mesh=mesh,
        compiler_params=pltpu.CompilerParams(needs_layout_passes=False),
        out_type=jax.ShapeDtypeStruct((_B,), jnp.float32),
        scratch_types=[
            pltpu.VMEM((_BPW,), jnp.int32),
            pltpu.VMEM((_BPW,), jnp.int32),
            pltpu.VMEM((_BPW,), jnp.int32),
            pltpu.VMEM((_BPW,), jnp.int32),
            [pltpu.VMEM((_CH,), jnp.int32) for _ in range(_NCHUNK)],
            [pltpu.VMEM((_CH,), jnp.int32) for _ in range(_NCHUNK)],
            pltpu.VMEM((_CH, 128), jnp.float32),
            pltpu.VMEM((_CH, 128), jnp.float32),
            pltpu.VMEM((_BPW * _ROWW,), jnp.float32),
            pltpu.VMEM((_BPW * _NCTX,), jnp.float32),
            pltpu.VMEM((_NWTS,), jnp.float32),
            pltpu.VMEM((_BPW,), jnp.float32),
            pltpu.SemaphoreType.DMA,
            pltpu.SemaphoreType.DMA,
        ],
    )
    def k(tu_h, ti_h, us_h, it_h, cx_h, wt_h, out_h,
          uidx, iidx, uoffs, ioffs, urows, irows, ubig, ibig,
          uic, cbuf, wbuf, obuf, usem, isem):
        wid = lax.axis_index("s") * _NC + lax.axis_index("c")
        base = wid * _BPW
        pltpu.sync_copy(us_h.at[pl.ds(base, _BPW)], uidx)
        pltpu.sync_copy(it_h.at[pl.ds(base, _BPW)], iidx)
        pltpu.sync_copy(cx_h.at[pl.ds(base * _NCTX, _BPW * _NCTX)], cbuf)
        pltpu.sync_copy(wt_h, wbuf)
        iota = lax.iota(jnp.int32, _L)

        for c in range(_NCHUNK):
            def rowoff(g, _, c=c):
                p = c * _CH + g * _L
                u = uidx[pl.ds(p, _L)]
                i = iidx[pl.ds(p, _L)]
                urows[c][pl.ds(g * _L, _L)] = u >> 3
                irows[c][pl.ds(g * _L, _L)] = i >> 3
                uoffs[pl.ds(p, _L)] = u & 7
                ioffs[pl.ds(p, _L)] = i & 7
                return _
            lax.fori_loop(0, _CH // _L, rowoff, None)

        for c in range(_NCHUNK):
            cu = pltpu.async_copy(tu_h.at[urows[c]], ubig, usem)
            ci = pltpu.async_copy(ti_h.at[irows[c]], ibig, isem)

            def extract(g, _, c=c, which=0):
                big, offs, col0 = ((ubig, uoffs, 0), (ibig, ioffs, _EMB))[which]
                rowv = g * _L + iota
                offv = offs[pl.ds(c * _CH + g * _L, _L)]
                colb = offv * _EMB
                flatb = (c * _CH + g * _L + iota) * _ROWW + col0
                for kk in range(_EMB):
                    vals = plsc.load_gather(big, [rowv, colb + kk])
                    plsc.store_scatter(uic, [flatb + kk], vals)
                return _

            cu.wait()
            lax.fori_loop(0, _CH // _L, functools.partial(extract, which=0),
                          None)
            ci.wait()
            lax.fori_loop(0, _CH // _L, functools.partial(extract, which=1),
                          None)

        def ctx_extract(g, _):
            rowv = g * _L + iota
            srcb = rowv * _NCTX
            dstb = rowv * _ROWW + 2 * _EMB
            for j in range(_NCTX):
                vals = plsc.load_gather(cbuf, [srcb + j])
                plsc.store_scatter(uic, [dstb + j], vals)
            return _
        lax.fori_loop(0, _BPW // _L, ctx_extract, None)

        w1a = [wbuf[pl.ds(r * _HID, _L)] for r in range(_ROWW)]
        w1b = [wbuf[pl.ds(r * _HID + _L, _L)] for r in range(_ROWW)]
        b1a = wbuf[pl.ds(_ROWW * _HID, _L)]
        b1b = wbuf[pl.ds(_ROWW * _HID + _L, _L)]
        w2a = wbuf[pl.ds(_ROWW * _HID + _HID, _L)]
        w2b = wbuf[pl.ds(_ROWW * _HID + _HID + _L, _L)]
        b2v = wbuf[pl.ds(_ROWW * _HID + 2 * _HID, _L)]
        lane15 = iota == (_L - 1)

        def mlp(s, _):
            b = s * _ROWW
            v0 = uic[pl.ds(b, _L)]
            v1 = uic[pl.ds(b + _L, _L)]
            v2 = uic[pl.ds(b + 24, _L)]
            acc0, acc1 = b1a, b1b
            for kk in range(_L):
                sp = _splat(v0, kk)
                acc0 = acc0 + sp * w1a[kk]
                acc1 = acc1 + sp * w1b[kk]
            for kk in range(_L):
                sp = _splat(v1, kk)
                acc0 = acc0 + sp * w1a[_EMB + kk]
                acc1 = acc1 + sp * w1b[_EMB + kk]
            for kk in range(8, _L):
                sp = _splat(v2, kk)
                acc0 = acc0 + sp * w1a[24 + kk]
                acc1 = acc1 + sp * w1b[24 + kk]
            h0 = jnp.maximum(acc0, 0.0)
            h1 = jnp.maximum(acc1, 0.0)
            t = h0 * w2a + h1 * w2b + b2v
            tc = jnp.cumsum(t)
            sv = lax.broadcast_in_dim(s, (_L,), ())
            plsc.store_scatter(obuf, [sv], tc, mask=lane15)
            return _
        lax.fori_loop(0, _BPW, mlp, None)

        pltpu.sync_copy(obuf, out_h.at[pl.ds(base, _BPW)])

    return k(tu, ti, users, items, ctx_flat, wts)


def _repack_body(a_ref, i_ref, o_ref):
    o_ref[...] = jax.lax.dot_general(
        a_ref[...], i_ref[...], (((0,), (0,)), ((), ())),
        preferred_element_type=jnp.float32)


def _repack(table_t):
    n = table_t.shape[1]
    blk = 8192
    grid = (n + blk - 1) // blk
    return pl.pallas_call(
        _repack_body,
        grid=(grid,),
        in_specs=[
            pl.BlockSpec((_EMB, blk), lambda j: (0, j)),
            pl.BlockSpec((_EMB, _EMB), lambda j: (0, 0)),
        ],
        out_specs=pl.BlockSpec((blk, _EMB), lambda j: (j, 0)),
        out_shape=jax.ShapeDtypeStruct((n, _EMB), jnp.float32),
    )(table_t, jnp.eye(_EMB, dtype=jnp.float32))


def kernel(users, items, context_features, user_table, item_table,
           ctx_W, ctx_b, W1, b1, W2, b2):
    tu = _repack(user_table.T).reshape(-1, 128)
    ti = _repack(item_table.T).reshape(-1, 128)
    cx = context_features.reshape(-1)
    w1c = W1[2 * _EMB:, :]
    w1f = jnp.concatenate([W1[:2 * _EMB, :], ctx_W @ w1c], axis=0)
    b1f = b1 + ctx_b @ w1c
    wts = jnp.concatenate([
        w1f.reshape(-1), b1f, W2[:, 0],
        jnp.pad(b2, (0, 15)),
    ])
    return _fused_sc(tu, ti, users.astype(jnp.int32), items.astype(jnp.int32),
                     cx, wts)


# speedup vs baseline: 1.4121x; 1.4121x over previous
import functools

import jax
import jax.numpy as jnp
from jax import lax
from jax.experimental import pallas as pl
from jax.experimental.pallas import tpu as pltpu
from jax.experimental.pallas import tpu_sc as plsc

_B = 16384
_EMB = 16
_HID = 32
_NCTX = 8
_NC = 2
_NS = 16
_NW = _NC * _NS
_BPW = _B // _NW
_CH = 256
_NCHUNK = _BPW // _CH
_L = 16


def _sc_gather(tu, ti, users, items):
    mesh = plsc.VectorSubcoreMesh(core_axis_name="c", subcore_axis_name="s")

    @functools.partial(
        pl.kernel,
        mesh=mesh,
        compiler_params=pltpu.CompilerParams(needs_layout_passes=False),
        out_type=[
            jax.ShapeDtypeStruct((_B, 128), jnp.float32),
            jax.ShapeDtypeStruct((_B, 128), jnp.float32),
        ],
        scratch_types=[
            pltpu.VMEM((_BPW,), jnp.int32),
            pltpu.VMEM((_BPW,), jnp.int32),
            [pltpu.VMEM((_CH,), jnp.int32) for _ in range(_NCHUNK)],
            [pltpu.VMEM((_CH,), jnp.int32) for _ in range(_NCHUNK)],
            pltpu.VMEM((_CH, 128), jnp.float32),
            pltpu.VMEM((_CH, 128), jnp.float32),
            pltpu.SemaphoreType.DMA,
            pltpu.SemaphoreType.DMA,
        ],
    )
    def k(tu_h, ti_h, us_h, it_h, ug_h, ig_h,
          uidx, iidx, urows, irows, ubig, ibig, usem, isem):
        wid = lax.axis_index("s") * _NC + lax.axis_index("c")
        base = wid * _BPW
        pltpu.sync_copy(us_h.at[pl.ds(base, _BPW)], uidx)
        pltpu.sync_copy(it_h.at[pl.ds(base, _BPW)], iidx)

        for c in range(_NCHUNK):
            def rowoff(g, _, c=c):
                p = c * _CH + g * _L
                urows[c][pl.ds(g * _L, _L)] = uidx[pl.ds(p, _L)] >> 3
                irows[c][pl.ds(g * _L, _L)] = iidx[pl.ds(p, _L)] >> 3
                return _
            lax.fori_loop(0, _CH // _L, rowoff, None)

        for c in range(_NCHUNK):
            cu = pltpu.async_copy(tu_h.at[urows[c]], ubig, usem)
            ci = pltpu.async_copy(ti_h.at[irows[c]], ibig, isem)
            cu.wait()
            pltpu.sync_copy(ubig, ug_h.at[pl.ds(base + c * _CH, _CH)])
            ci.wait()
            pltpu.sync_copy(ibig, ig_h.at[pl.ds(base + c * _CH, _CH)])

    return k(tu, ti, users, items)


_TB = 2048


def _mlp_body(ug_ref, ig_ref, us_ref, it_ref, cx_ref,
              w1u_ref, w1i_ref, w1c_ref, b1_ref, w2_ref, o_ref):
    lane = lax.broadcasted_iota(jnp.int32, (_TB, 128), 1) >> 4
    uoff = lax.broadcast_in_dim(us_ref[...] & 7, (_TB, 128), (0,))
    ioff = lax.broadcast_in_dim(it_ref[...] & 7, (_TB, 128), (0,))
    xu = jnp.where(lane == uoff, ug_ref[...], 0.0)
    xi = jnp.where(lane == ioff, ig_ref[...], 0.0)
    h = (jax.lax.dot_general(xu, w1u_ref[...], (((1,), (0,)), ((), ())),
                             preferred_element_type=jnp.float32)
         + jax.lax.dot_general(xi, w1i_ref[...], (((1,), (0,)), ((), ())),
                               preferred_element_type=jnp.float32)
         + jax.lax.dot_general(cx_ref[...], w1c_ref[...],
                               (((1,), (0,)), ((), ())),
                               preferred_element_type=jnp.float32)
         + b1_ref[...])
    h = jnp.maximum(h, 0.0)
    o_ref[...] = jax.lax.dot_general(
        h, w2_ref[...], (((1,), (0,)), ((), ())),
        preferred_element_type=jnp.float32)[:, 0]


def _tc_mlp(ug, ig, users, items, ctx, w1u, w1i, w1c, b1f, w2):
    grid = (_B // _TB,)
    return pl.pallas_call(
        _mlp_body,
        grid=grid,
        in_specs=[
            pl.BlockSpec((_TB, 128), lambda j: (j, 0)),
            pl.BlockSpec((_TB, 128), lambda j: (j, 0)),
            pl.BlockSpec((_TB,), lambda j: (j,)),
            pl.BlockSpec((_TB,), lambda j: (j,)),
            pl.BlockSpec((_TB, _NCTX), lambda j: (j, 0)),
            pl.BlockSpec((128, _HID), lambda j: (0, 0)),
            pl.BlockSpec((128, _HID), lambda j: (0, 0)),
            pl.BlockSpec((_NCTX, _HID), lambda j: (0, 0)),
            pl.BlockSpec((1, _HID), lambda j: (0, 0)),
            pl.BlockSpec((_HID, 1), lambda j: (0, 0)),
        ],
        out_specs=pl.BlockSpec((_TB,), lambda j: (j,)),
        out_shape=jax.ShapeDtypeStruct((_B,), jnp.float32),
    )(ug, ig, users, items, ctx, w1u, w1i, w1c, b1f, w2)


def kernel(users, items, context_features, user_table, item_table,
           ctx_W, ctx_b, W1, b1, W2, b2):
    tu = user_table.reshape(-1, 128)
    ti = item_table.reshape(-1, 128)
    us = users.astype(jnp.int32)
    it = items.astype(jnp.int32)
    ug, ig = _sc_gather(tu, ti, us, it)
    w1c = ctx_W @ W1[2 * _EMB:, :]
    b1f = (b1 + ctx_b @ W1[2 * _EMB:, :])[None, :]
    w1u = jnp.tile(W1[:_EMB, :], (8, 1))
    w1i = jnp.tile(W1[_EMB:2 * _EMB, :], (8, 1))
    w2 = W2 + 0.0
    out = _tc_mlp(ug, ig, us, it, context_features, w1u, w1i, w1c, b1f, w2)
    return out + b2[0]


# speedup vs baseline: 1.4190x; 1.0049x over previous
import functools

import jax
import jax.numpy as jnp
from jax import lax
from jax.experimental import pallas as pl
from jax.experimental.pallas import tpu as pltpu
from jax.experimental.pallas import tpu_sc as plsc

_B = 16384
_EMB = 16
_HID = 32
_NCTX = 8
_NC = 2
_NS = 16
_NW = _NC * _NS
_BPW = _B // _NW
_CH = 256
_NCHUNK = _BPW // _CH
_L = 16


def _sc_gather(table, idx):
    mesh = plsc.VectorSubcoreMesh(core_axis_name="c", subcore_axis_name="s")

    @functools.partial(
        pl.kernel,
        mesh=mesh,
        compiler_params=pltpu.CompilerParams(needs_layout_passes=False),
        out_type=jax.ShapeDtypeStruct((_B, 128), jnp.float32),
        scratch_types=[
            pltpu.VMEM((_BPW,), jnp.int32),
            [pltpu.VMEM((_CH,), jnp.int32) for _ in range(_NCHUNK)],
            [pltpu.VMEM((_CH, 128), jnp.float32) for _ in range(_NCHUNK)],
            [pltpu.SemaphoreType.DMA for _ in range(_NCHUNK)],
        ],
    )
    def k(t_h, idx_h, g_h, idxv, rows, big, sems):
        wid = lax.axis_index("s") * _NC + lax.axis_index("c")
        base = wid * _BPW

        pltpu.sync_copy(idx_h.at[pl.ds(base, _BPW)], idxv)
        for c in range(_NCHUNK):
            def rowoff(g, _, c=c):
                p = c * _CH + g * _L
                rows[c][pl.ds(g * _L, _L)] = idxv[pl.ds(p, _L)] >> 3
                return _
            lax.fori_loop(0, _CH // _L, rowoff, None)

        cps = [pltpu.async_copy(t_h.at[rows[c]], big[c], sems[c])
               for c in range(_NCHUNK)]
        for c in range(_NCHUNK):
            cps[c].wait()
            pltpu.sync_copy(big[c], g_h.at[pl.ds(base + c * _CH, _CH)])

    return k(table, idx)


_TB = 2048


def _mlp_body(ug_ref, ig_ref, us_ref, it_ref, cx_ref,
              w1u_ref, w1i_ref, w1c_ref, b1_ref, w2_ref, o_ref):
    lane = lax.broadcasted_iota(jnp.int32, (_TB, 128), 1) >> 4
    uoff = lax.broadcast_in_dim(us_ref[...] & 7, (_TB, 128), (0,))
    ioff = lax.broadcast_in_dim(it_ref[...] & 7, (_TB, 128), (0,))
    xu = jnp.where(lane == uoff, ug_ref[...], 0.0)
    xi = jnp.where(lane == ioff, ig_ref[...], 0.0)
    h = (jax.lax.dot_general(xu, w1u_ref[...], (((1,), (0,)), ((), ())),
                             preferred_element_type=jnp.float32)
         + jax.lax.dot_general(xi, w1i_ref[...], (((1,), (0,)), ((), ())),
                               preferred_element_type=jnp.float32)
         + jax.lax.dot_general(cx_ref[...], w1c_ref[...],
                               (((1,), (0,)), ((), ())),
                               preferred_element_type=jnp.float32)
         + b1_ref[...])
    h = jnp.maximum(h, 0.0)
    o_ref[...] = jax.lax.dot_general(
        h, w2_ref[...], (((1,), (0,)), ((), ())),
        preferred_element_type=jnp.float32)[:, 0]


def _tc_mlp(ug, ig, users, items, ctx, w1u, w1i, w1c, b1f, w2):
    grid = (_B // _TB,)
    return pl.pallas_call(
        _mlp_body,
        grid=grid,
        in_specs=[
            pl.BlockSpec((_TB, 128), lambda j: (j, 0)),
            pl.BlockSpec((_TB, 128), lambda j: (j, 0)),
            pl.BlockSpec((_TB,), lambda j: (j,)),
            pl.BlockSpec((_TB,), lambda j: (j,)),
            pl.BlockSpec((_TB, _NCTX), lambda j: (j, 0)),
            pl.BlockSpec((128, _HID), lambda j: (0, 0)),
            pl.BlockSpec((128, _HID), lambda j: (0, 0)),
            pl.BlockSpec((_NCTX, _HID), lambda j: (0, 0)),
            pl.BlockSpec((1, _HID), lambda j: (0, 0)),
            pl.BlockSpec((_HID, 1), lambda j: (0, 0)),
        ],
        out_specs=pl.BlockSpec((_TB,), lambda j: (j,)),
        out_shape=jax.ShapeDtypeStruct((_B,), jnp.float32),
    )(ug, ig, users, items, ctx, w1u, w1i, w1c, b1f, w2)


def kernel(users, items, context_features, user_table, item_table,
           ctx_W, ctx_b, W1, b1, W2, b2):
    tu = user_table.reshape(-1, 128)
    ti = item_table.reshape(-1, 128)
    us = users.astype(jnp.int32)
    it = items.astype(jnp.int32)
    ug = _sc_gather(tu, us)
    ig = _sc_gather(ti, it)
    w1c = ctx_W @ W1[2 * _EMB:, :]
    b1f = (b1 + ctx_b @ W1[2 * _EMB:, :])[None, :]
    w1u = jnp.tile(W1[:_EMB, :], (8, 1))
    w1i = jnp.tile(W1[_EMB:2 * _EMB, :], (8, 1))
    w2 = W2 + 0.0
    out = _tc_mlp(ug, ig, us, it, context_features, w1u, w1i, w1c, b1f, w2)
    return out + b2[0]


# speedup vs baseline: 2.0015x; 1.4105x over previous
import functools

import jax
import jax.numpy as jnp
from jax import lax
from jax.experimental import pallas as pl
from jax.experimental.pallas import tpu as pltpu
from jax.experimental.pallas import tpu_sc as plsc

_B = 16384
_EMB = 16
_HID = 32
_NCTX = 8
_NC = 2
_NS = 16
_NW = _NC * _NS
_BPW = _B // _NW
_CH = 256
_NCHUNK = _BPW // _CH
_L = 16


def _sc_gather(table, idx):
    mesh = plsc.VectorSubcoreMesh(core_axis_name="c", subcore_axis_name="s")

    @functools.partial(
        pl.kernel,
        mesh=mesh,
        compiler_params=pltpu.CompilerParams(needs_layout_passes=False),
        out_type=jax.ShapeDtypeStruct((_B, 128), jnp.float32),
        scratch_types=[
            pltpu.VMEM((_BPW,), jnp.int32),
            [pltpu.VMEM((_CH,), jnp.int32) for _ in range(_NCHUNK)],
            [pltpu.VMEM((_CH, 128), jnp.float32) for _ in range(_NCHUNK)],
            [pltpu.SemaphoreType.DMA for _ in range(_NCHUNK)],
        ],
    )
    def k(t_h, idx_h, g_h, idxv, rows, big, sems):
        wid = lax.axis_index("s") * _NC + lax.axis_index("c")
        base = wid * _BPW

        pltpu.sync_copy(idx_h.at[pl.ds(base, _BPW)], idxv)
        for c in range(_NCHUNK):
            def rowoff(g, _, c=c):
                p = c * _CH + g * _L
                rows[c][pl.ds(g * _L, _L)] = idxv[pl.ds(p, _L)] >> 3
                return _
            lax.fori_loop(0, _CH // _L, rowoff, None)

        cps = [pltpu.async_copy(t_h.at[rows[c]], big[c], sems[c])
               for c in range(_NCHUNK)]
        for c in range(_NCHUNK):
            cps[c].wait()
            pltpu.sync_copy(big[c], g_h.at[pl.ds(base + c * _CH, _CH)])

    return k(table, idx)


_TB = 2048


def _mlp_body(ug_ref, ig_ref, us_ref, it_ref, cx_ref,
              w1u_ref, w1i_ref, w1c_ref, b1_ref, w2_ref, o_ref):
    lane = lax.broadcasted_iota(jnp.int32, (_TB, 128), 1) >> 4
    uoff = lax.broadcast_in_dim(us_ref[...] & 7, (_TB, 128), (0,))
    ioff = lax.broadcast_in_dim(it_ref[...] & 7, (_TB, 128), (0,))
    xu = jnp.where(lane == uoff, ug_ref[...], 0.0)
    xi = jnp.where(lane == ioff, ig_ref[...], 0.0)
    h = (jax.lax.dot_general(xu, w1u_ref[...], (((1,), (0,)), ((), ())),
                             preferred_element_type=jnp.float32)
         + jax.lax.dot_general(xi, w1i_ref[...], (((1,), (0,)), ((), ())),
                               preferred_element_type=jnp.float32)
         + jax.lax.dot_general(cx_ref[...], w1c_ref[...],
                               (((1,), (0,)), ((), ())),
                               preferred_element_type=jnp.float32)
         + b1_ref[...])
    h = jnp.maximum(h, 0.0)
    o_ref[...] = jax.lax.dot_general(
        h, w2_ref[...], (((1,), (0,)), ((), ())),
        preferred_element_type=jnp.float32)[:, 0]


def _tc_mlp(ug, ig, users, items, ctx, w1u, w1i, w1c, b1f, w2):
    grid = (_B // _TB,)
    return pl.pallas_call(
        _mlp_body,
        grid=grid,
        in_specs=[
            pl.BlockSpec((_TB, 128), lambda j: (j, 0)),
            pl.BlockSpec((_TB, 128), lambda j: (j, 0)),
            pl.BlockSpec((_TB,), lambda j: (j,)),
            pl.BlockSpec((_TB,), lambda j: (j,)),
            pl.BlockSpec((_TB, _NCTX), lambda j: (j, 0)),
            pl.BlockSpec((128, _HID), lambda j: (0, 0)),
            pl.BlockSpec((128, _HID), lambda j: (0, 0)),
            pl.BlockSpec((_NCTX, _HID), lambda j: (0, 0)),
            pl.BlockSpec((1, _HID), lambda j: (0, 0)),
            pl.BlockSpec((_HID, 1), lambda j: (0, 0)),
        ],
        out_specs=pl.BlockSpec((_TB,), lambda j: (j,)),
        out_shape=jax.ShapeDtypeStruct((_B,), jnp.float32),
    )(ug, ig, users, items, ctx, w1u, w1i, w1c, b1f, w2)


_RR = 1024


def _repack_body(a_ref, i_ref, o_ref):
    t = jax.lax.dot_general(
        a_ref[...], i_ref[...], (((0,), (0,)), ((), ())),
        preferred_element_type=jnp.float32)
    t3 = t.reshape(_RR, 8, _EMB)
    for o in range(8):
        o_ref[:, o * _EMB:(o + 1) * _EMB] = t3[:, o, :]


def _repack(table_t):
    n = table_t.shape[1]
    grid = ((n + 8 * _RR - 1) // (8 * _RR),)
    return pl.pallas_call(
        _repack_body,
        grid=grid,
        in_specs=[
            pl.BlockSpec((_EMB, 8 * _RR), lambda j: (0, j)),
            pl.BlockSpec((_EMB, _EMB), lambda j: (0, 0)),
        ],
        out_specs=pl.BlockSpec((_RR, 128), lambda j: (j, 0)),
        out_shape=jax.ShapeDtypeStruct((n // 8, 128), jnp.float32),
    )(table_t, jnp.eye(_EMB, dtype=jnp.float32))


def kernel(users, items, context_features, user_table, item_table,
           ctx_W, ctx_b, W1, b1, W2, b2):
    tu = _repack(user_table.T)
    ti = _repack(item_table.T)
    us = users.astype(jnp.int32)
    it = items.astype(jnp.int32)
    ug = _sc_gather(tu, us)
    ig = _sc_gather(ti, it)
    w1c = ctx_W @ W1[2 * _EMB:, :]
    b1f = (b1 + ctx_b @ W1[2 * _EMB:, :])[None, :]
    w1u = jnp.tile(W1[:_EMB, :], (8, 1))
    w1i = jnp.tile(W1[_EMB:2 * _EMB, :], (8, 1))
    w2 = W2 + 0.0
    out = _tc_mlp(ug, ig, us, it, context_features, w1u, w1i, w1c, b1f, w2)
    return out + b2[0]
